# baseline (speedup 1.0000x reference)
import jax
import jax.numpy as jnp
from jax import lax
from jax.experimental import pallas as pl
from jax.experimental.pallas import tpu as pltpu
from jax.experimental.pallas import tpu_sc as plsc

N = 320000
D = 128
M = 4096

NC = 2
NS = 16
NW = NC * NS
ROWS_PER_TILE = N // NW
BLK = 80
NBLK = ROWS_PER_TILE // BLK
STRIPE = M // NS
CH = 64


def _sc_body(vi_hbm, idx_hbm, sums_hbm, counts_hbm,
             buf_a, buf_b, idx_all, ones_buf, acc_shared, cnt_shared,
             gsem_a, gsem_b, rsem, osem):
  core = lax.axis_index("c")
  sub = lax.axis_index("s")
  wid = core * NS + sub
  base = wid * ROWS_PER_TILE

  zeros16 = jnp.zeros((16,), jnp.float32)
  ones16 = jnp.ones((16,), jnp.float32)

  def _zero_row(i, _):
    for j in range(D // 16):
      buf_a[i, pl.ds(j * 16, 16)] = zeros16
    return 0
  lax.fori_loop(0, BLK, _zero_row, 0)

  def _fill_ones(i, _):
    for j in range(D // 16):
      ones_buf[i, pl.ds(j * 16, 16)] = ones16
    return 0
  lax.fori_loop(0, BLK, _fill_ones, 0)

  zsl = buf_a.at[pl.ds(0, CH)]
  for c in range(STRIPE // CH):
    pltpu.sync_copy(zsl, acc_shared.at[pl.ds(sub * STRIPE + c * CH, CH)])
    pltpu.sync_copy(zsl, cnt_shared.at[pl.ds(sub * STRIPE + c * CH, CH)])

  pltpu.sync_copy(idx_hbm.at[wid], idx_all)

  pltpu.async_copy(vi_hbm.at[pl.ds(base, BLK)], buf_a, gsem_a)
  pltpu.async_copy(vi_hbm.at[pl.ds(base + BLK, BLK)], buf_b, gsem_b)
  plsc.subcore_barrier()

  def _half(s, buf, gsem):
    pltpu.make_async_copy(
        vi_hbm.at[pl.ds(base + s * BLK, BLK)], buf, gsem).wait()
    iv = idx_all.at[s]
    cp1 = pltpu.async_copy(buf, acc_shared.at[iv], rsem, add=True)
    pltpu.async_copy(ones_buf, cnt_shared.at[iv], osem, add=True)
    @pl.when(s >= 1)
    def _():
      pltpu.make_async_copy(ones_buf, cnt_shared.at[iv], osem).wait()
    cp1.wait()
    @pl.when(s + 2 < NBLK)
    def _():
      pltpu.async_copy(vi_hbm.at[pl.ds(base + (s + 2) * BLK, BLK)], buf,
                       gsem)

  def _pair(t, _):
    _half(2 * t, buf_a, gsem_a)
    _half(2 * t + 1, buf_b, gsem_b)
    return 0
  lax.fori_loop(0, (NBLK - 1) // 2, _pair, 0)
  _half(NBLK - 1, buf_a, gsem_a)
  pltpu.make_async_copy(ones_buf, cnt_shared.at[idx_all.at[0]],
                        osem).wait()

  plsc.subcore_barrier()
  asl = buf_a.at[pl.ds(0, CH)]
  csl = buf_b.at[pl.ds(0, CH)]
  for c in range(STRIPE // CH):
    srow = sub * STRIPE + c * CH
    drow = core * M + srow
    pltpu.sync_copy(acc_shared.at[pl.ds(srow, CH)], asl)
    pltpu.sync_copy(asl, sums_hbm.at[pl.ds(drow, CH)])
    pltpu.sync_copy(cnt_shared.at[pl.ds(srow, CH)], csl)
    pltpu.sync_copy(csl, counts_hbm.at[pl.ds(drow, CH)])


@jax.jit
def _sc_segsum(vi, idx):
  mesh = plsc.VectorSubcoreMesh(
      core_axis_name="c", subcore_axis_name="s", num_cores=NC,
      num_subcores=NS)
  f = pl.kernel(
      _sc_body,
      out_type=(
          jax.ShapeDtypeStruct((NC * M, D), jnp.float32),
          jax.ShapeDtypeStruct((NC * M, D), jnp.float32),
      ),
      mesh=mesh,
      scratch_types=[
          pltpu.VMEM((BLK, D), jnp.float32),
          pltpu.VMEM((BLK, D), jnp.float32),
          pltpu.VMEM((NBLK, BLK), jnp.int32),
          pltpu.VMEM((BLK, D), jnp.float32),
          pltpu.VMEM_SHARED((M, D), jnp.float32),
          pltpu.VMEM_SHARED((M, D), jnp.float32),
          pltpu.SemaphoreType.DMA,
          pltpu.SemaphoreType.DMA,
          pltpu.SemaphoreType.DMA,
          pltpu.SemaphoreType.DMA,
      ],
  )
  return f(vi, idx.reshape(NW, NBLK, BLK))


def _tc_body(parts_ref, counts_ref, w1_ref, b1_ref, g1_ref, be1_ref,
             w2_ref, b2_ref, g2_ref, be2_ref, w3_ref, b3_ref, out_ref):
  seg = parts_ref[0:M] + parts_ref[M:2 * M]
  cnt = counts_ref[0:M, 0:1] + counts_ref[M:2 * M, 0:1]
  mean = seg / jnp.maximum(cnt, 1.0)

  h = jnp.dot(mean, w1_ref[...], preferred_element_type=jnp.float32)
  h = h + b1_ref[...]
  mu = jnp.mean(h, axis=0, keepdims=True)
  var = jnp.mean((h - mu) * (h - mu), axis=0, keepdims=True)
  h = (h - mu) / jnp.sqrt(var + 1e-5) * g1_ref[...] + be1_ref[...]
  h = jnp.maximum(h, 0.0)

  h = jnp.dot(h, w2_ref[...], preferred_element_type=jnp.float32)
  h = h + b2_ref[...]
  mu = jnp.mean(h, axis=0, keepdims=True)
  var = jnp.mean((h - mu) * (h - mu), axis=0, keepdims=True)
  h = (h - mu) / jnp.sqrt(var + 1e-5) * g2_ref[...] + be2_ref[...]
  h = jnp.maximum(h, 0.0)

  out_ref[...] = (
      jnp.dot(h, w3_ref[...], preferred_element_type=jnp.float32)
      + b3_ref[...])


@jax.jit
def _tc_mlp(parts, counts, W1, b1, g1, be1, W2, b2, g2, be2, W3, b3):
  return pl.pallas_call(
      _tc_body,
      out_shape=jax.ShapeDtypeStruct((M, 1), jnp.float32),
  )(parts, counts, W1, b1, g1, be1, W2, b2, g2, be2, W3, b3)


def kernel(vi, atom_mol_batch, W1, b1, g1, be1, W2, b2, g2, be2, W3, b3):
  idx = atom_mol_batch.astype(jnp.int32)
  parts, counts = _sc_segsum(vi, idx)
  return _tc_mlp(
      parts, counts, W1, b1.reshape(1, D), g1.reshape(1, D),
      be1.reshape(1, D), W2, b2.reshape(1, D), g2.reshape(1, D),
      be2.reshape(1, D), W3, b3.reshape(1, 1))


# speedup vs baseline: 1.0564x; 1.0564x over previous
import jax
import jax.numpy as jnp
from jax import lax
from jax.experimental import pallas as pl
from jax.experimental.pallas import tpu as pltpu
from jax.experimental.pallas import tpu_sc as plsc

N = 320000
D = 128
M = 4096

NC = 2
NS = 16
NW = NC * NS
ROWS_PER_TILE = N // NW
BLK = 80
NBLK = ROWS_PER_TILE // BLK
STRIPE = M // NS
CH = 64
SEGS = M // NW
NR = N // D
NRP = 2560
RMP = NRP + 16
BIG = 1 << 30


def _sc_body(vi_hbm, idx_hbm, idx2d_hbm, rowmax_hbm, sums_hbm, counts_hbm,
             buf_a, buf_b, idx_all, rowmax_buf, region_buf, cnt_buf,
             acc_shared, gsem_a, gsem_b, rsem):
  core = lax.axis_index("c")
  sub = lax.axis_index("s")
  wid = core * NS + sub
  base = wid * ROWS_PER_TILE

  zeros16 = jnp.zeros((16,), jnp.float32)

  def _zero_row(i, _):
    for j in range(D // 16):
      buf_a[i, pl.ds(j * 16, 16)] = zeros16
    return 0
  lax.fori_loop(0, BLK, _zero_row, 0)

  zsl = buf_a.at[pl.ds(0, CH)]
  for c in range(STRIPE // CH):
    pltpu.sync_copy(zsl, acc_shared.at[pl.ds(sub * STRIPE + c * CH, CH)])

  pltpu.sync_copy(idx_hbm.at[wid], idx_all)

  pltpu.async_copy(vi_hbm.at[pl.ds(base, BLK)], buf_a, gsem_a)
  pltpu.async_copy(vi_hbm.at[pl.ds(base + BLK, BLK)], buf_b, gsem_b)
  plsc.subcore_barrier()

  def _half(s, buf, gsem):
    pltpu.make_async_copy(
        vi_hbm.at[pl.ds(base + s * BLK, BLK)], buf, gsem).wait()
    pltpu.async_copy(buf, acc_shared.at[idx_all.at[s]], rsem,
                     add=True).wait()
    @pl.when(s + 2 < NBLK)
    def _():
      pltpu.async_copy(vi_hbm.at[pl.ds(base + (s + 2) * BLK, BLK)], buf,
                       gsem)

  def _pair(t, _):
    _half(2 * t, buf_a, gsem_a)
    _half(2 * t + 1, buf_b, gsem_b)
    return 0
  lax.fori_loop(0, (NBLK - 1) // 2, _pair, 0)
  _half(NBLK - 1, buf_a, gsem_a)

  pltpu.sync_copy(rowmax_hbm, rowmax_buf)
  m0 = wid * SEGS
  lane = lax.iota(jnp.int32, 16)

  def _pcount(v, m):
    t = jnp.int32(0)
    for j in range(16):
      t = t + jnp.where(v[j] <= m, 1, 0)
    return t

  def _find_row(m):
    def bs(_, lohi):
      lo, hi = lohi
      mid = (lo + hi) // 2
      v = rowmax_buf[pl.ds(pl.multiple_of(mid * 8, 8), 16)]
      go = v[0] <= m
      nlo = jnp.where(go, mid + 1, lo)
      nhi = jnp.where(go, hi, mid)
      live = lo < hi
      return (jnp.where(live, nlo, lo), jnp.where(live, nhi, hi))
    p, _ = lax.fori_loop(0, 9, bs, (jnp.int32(0), jnp.int32(NRP // 8)))
    bl = jnp.maximum(8 * (p - 1), 0)
    w = rowmax_buf[pl.ds(pl.multiple_of(bl, 8), 16)]
    return bl + _pcount(w, m)

  def _e_of(m, cbase):
    r = _find_row(m)
    need = (r < cbase) | (r >= cbase + 128)
    ncbase = jnp.where(need, jnp.minimum((r // 8) * 8, NRP - 128), cbase)

    @pl.when(need)
    def _():
      pltpu.sync_copy(
          idx2d_hbm.at[pl.ds(pl.multiple_of(ncbase, 8), 128)], region_buf)
    rr = r - ncbase

    def ibs(_, lohi):
      lo, hi = lohi
      mid = (lo + hi) // 2
      v = region_buf[rr, pl.ds(pl.multiple_of(mid * 8, 8), 16)]
      go = v[0] <= m
      nlo = jnp.where(go, mid + 1, lo)
      nhi = jnp.where(go, hi, mid)
      live = lo < hi
      return (jnp.where(live, nlo, lo), jnp.where(live, nhi, hi))
    q, _ = lax.fori_loop(0, 4, ibs, (jnp.int32(0), jnp.int32(D // 8 - 1)))
    bl = jnp.clip(8 * (q - 1), 0, D - 16)
    w = region_buf[rr, pl.ds(pl.multiple_of(bl, 8), 16)]
    return r * D + bl + _pcount(w, m), ncbase

  e_prev0, cb0 = _e_of(m0 - 1, jnp.int32(-(1 << 20)))

  def _group(g, carry):
    e_prev, cbase = carry

    def _one(k, kc):
      e_prev, cbase, vec = kc
      m = m0 + g * 16 + k
      e, cbase = _e_of(m, cbase)
      d = (e - e_prev).astype(jnp.float32)
      vec = jnp.where(lane == k, d, vec)
      return (e, cbase, vec)

    e_prev, cbase, vec = lax.fori_loop(
        0, 16, _one, (e_prev, cbase, zeros16))
    cnt_buf[pl.ds(g * 16, 16)] = vec
    return (e_prev, cbase)

  lax.fori_loop(0, SEGS // 16, _group, (e_prev0, cb0))
  pltpu.sync_copy(cnt_buf, counts_hbm.at[wid])

  plsc.subcore_barrier()
  asl = buf_a.at[pl.ds(0, CH)]
  for c in range(STRIPE // CH):
    srow = sub * STRIPE + c * CH
    pltpu.sync_copy(acc_shared.at[pl.ds(srow, CH)], asl)
    pltpu.sync_copy(asl, sums_hbm.at[pl.ds(core * M + srow, CH)])


@jax.jit
def _sc_segsum(vi, idx):
  mesh = plsc.VectorSubcoreMesh(
      core_axis_name="c", subcore_axis_name="s", num_cores=NC,
      num_subcores=NS)
  f = pl.kernel(
      _sc_body,
      out_type=(
          jax.ShapeDtypeStruct((NC * M, D), jnp.float32),
          jax.ShapeDtypeStruct((NW, SEGS), jnp.float32),
      ),
      mesh=mesh,
      scratch_types=[
          pltpu.VMEM((BLK, D), jnp.float32),
          pltpu.VMEM((BLK, D), jnp.float32),
          pltpu.VMEM((NBLK, BLK), jnp.int32),
          pltpu.VMEM((RMP,), jnp.int32),
          pltpu.VMEM((128, D), jnp.int32),
          pltpu.VMEM((SEGS,), jnp.float32),
          pltpu.VMEM_SHARED((M, D), jnp.float32),
          pltpu.SemaphoreType.DMA,
          pltpu.SemaphoreType.DMA,
          pltpu.SemaphoreType.DMA,
      ],
  )
  idx_pad = jnp.concatenate(
      [idx, jnp.full((NRP * D - N,), BIG, jnp.int32)]).reshape(NRP, D)
  rowmax = jnp.concatenate(
      [idx_pad[:, D - 1], jnp.full((RMP - NRP,), BIG, jnp.int32)])
  return f(vi, idx.reshape(NW, NBLK, BLK), idx_pad, rowmax)


def _tc_body(parts_ref, counts_ref, w1_ref, b1_ref, g1_ref, be1_ref,
             w2_ref, b2_ref, g2_ref, be2_ref, w3_ref, b3_ref, out_ref):
  seg = parts_ref[0:M] + parts_ref[M:2 * M]
  cnt = counts_ref[...]
  mean = seg / jnp.maximum(cnt, 1.0)

  h = jnp.dot(mean, w1_ref[...], preferred_element_type=jnp.float32)
  h = h + b1_ref[...]
  mu = jnp.mean(h, axis=0, keepdims=True)
  var = jnp.mean((h - mu) * (h - mu), axis=0, keepdims=True)
  h = (h - mu) / jnp.sqrt(var + 1e-5) * g1_ref[...] + be1_ref[...]
  h = jnp.maximum(h, 0.0)

  h = jnp.dot(h, w2_ref[...], preferred_element_type=jnp.float32)
  h = h + b2_ref[...]
  mu = jnp.mean(h, axis=0, keepdims=True)
  var = jnp.mean((h - mu) * (h - mu), axis=0, keepdims=True)
  h = (h - mu) / jnp.sqrt(var + 1e-5) * g2_ref[...] + be2_ref[...]
  h = jnp.maximum(h, 0.0)

  out_ref[...] = (
      jnp.dot(h, w3_ref[...], preferred_element_type=jnp.float32)
      + b3_ref[...])


@jax.jit
def _tc_mlp(parts, counts, W1, b1, g1, be1, W2, b2, g2, be2, W3, b3):
  return pl.pallas_call(
      _tc_body,
      out_shape=jax.ShapeDtypeStruct((M, 1), jnp.float32),
  )(parts, counts, W1, b1, g1, be1, W2, b2, g2, be2, W3, b3)


def kernel(vi, atom_mol_batch, W1, b1, g1, be1, W2, b2, g2, be2, W3, b3):
  idx = atom_mol_batch.astype(jnp.int32)
  parts, counts = _sc_segsum(vi, idx)
  return _tc_mlp(
      parts, counts.reshape(M, 1), W1, b1.reshape(1, D), g1.reshape(1, D),
      be1.reshape(1, D), W2, b2.reshape(1, D), g2.reshape(1, D),
      be2.reshape(1, D), W3, b3.reshape(1, 1))


# speedup vs baseline: 1.0835x; 1.0257x over previous
import jax
import jax.numpy as jnp
from jax import lax
from jax.experimental import pallas as pl
from jax.experimental.pallas import tpu as pltpu
from jax.experimental.pallas import tpu_sc as plsc

N = 320000
D = 128
M = 4096

NC = 2
NS = 16
NW = NC * NS
ROWS_PER_TILE = N // NW
BLK = 80
NBLK = ROWS_PER_TILE // BLK
STRIPE = M // NS
CH = 64
SEGS = M // NW
NR = N // D
NRP = 2560
RMP = NRP + 16
BIG = 1 << 30


def _sc_body(vi_hbm, idx_hbm, idx2d_hbm, rowmax_hbm, sums_hbm, counts_hbm,
             buf_a, buf_b, idx_all, rowmax_buf, region_buf, cnt_buf,
             acc_shared, gsem_a, gsem_b, rsem):
  core = lax.axis_index("c")
  sub = lax.axis_index("s")
  wid = core * NS + sub
  base = wid * ROWS_PER_TILE

  zeros16 = jnp.zeros((16,), jnp.float32)

  def _zero_row(i, _):
    for j in range(D // 16):
      buf_a[i, pl.ds(j * 16, 16)] = zeros16
    return 0
  lax.fori_loop(0, BLK, _zero_row, 0)

  zsl = buf_a.at[pl.ds(0, CH)]
  for c in range(STRIPE // CH):
    pltpu.sync_copy(zsl, acc_shared.at[pl.ds(sub * STRIPE + c * CH, CH)])

  pltpu.sync_copy(idx_hbm.at[wid], idx_all)

  pltpu.async_copy(vi_hbm.at[pl.ds(base, BLK)], buf_a, gsem_a)
  pltpu.async_copy(vi_hbm.at[pl.ds(base + BLK, BLK)], buf_b, gsem_b)
  pltpu.sync_copy(rowmax_hbm, rowmax_buf)
  plsc.subcore_barrier()

  def _half(s, buf, gsem):
    pltpu.make_async_copy(
        vi_hbm.at[pl.ds(base + s * BLK, BLK)], buf, gsem).wait()
    pltpu.async_copy(buf, acc_shared.at[idx_all.at[s]], rsem,
                     add=True).wait()
    @pl.when(s + 2 < NBLK)
    def _():
      pltpu.async_copy(vi_hbm.at[pl.ds(base + (s + 2) * BLK, BLK)], buf,
                       gsem)

  m0 = wid * SEGS
  lane = lax.iota(jnp.int32, 16)

  def _pcount(v, m):
    t = jnp.int32(0)
    for j in range(16):
      t = t + jnp.where(v[j] <= m, 1, 0)
    return t

  def _find_row(m):
    def bs(_, lohi):
      lo, hi = lohi
      mid = (lo + hi) // 2
      v = rowmax_buf[pl.ds(pl.multiple_of(mid * 8, 8), 16)]
      go = v[0] <= m
      nlo = jnp.where(go, mid + 1, lo)
      nhi = jnp.where(go, hi, mid)
      live = lo < hi
      return (jnp.where(live, nlo, lo), jnp.where(live, nhi, hi))
    p, _ = lax.fori_loop(0, 9, bs, (jnp.int32(0), jnp.int32(NRP // 8)))
    bl = jnp.maximum(8 * (p - 1), 0)
    w = rowmax_buf[pl.ds(pl.multiple_of(bl, 8), 16)]
    return bl + _pcount(w, m)

  def _e_of(m, cbase):
    r = _find_row(m)
    need = (r < cbase) | (r >= cbase + 128)
    ncbase = jnp.where(need, jnp.minimum((r // 8) * 8, NRP - 128), cbase)

    @pl.when(need)
    def _():
      pltpu.sync_copy(
          idx2d_hbm.at[pl.ds(pl.multiple_of(ncbase, 8), 128)], region_buf)
    rr = r - ncbase

    def ibs(_, lohi):
      lo, hi = lohi
      mid = (lo + hi) // 2
      v = region_buf[rr, pl.ds(pl.multiple_of(mid * 8, 8), 16)]
      go = v[0] <= m
      nlo = jnp.where(go, mid + 1, lo)
      nhi = jnp.where(go, hi, mid)
      live = lo < hi
      return (jnp.where(live, nlo, lo), jnp.where(live, nhi, hi))
    q, _ = lax.fori_loop(0, 4, ibs, (jnp.int32(0), jnp.int32(D // 8 - 1)))
    bl = jnp.clip(8 * (q - 1), 0, D - 16)
    w = region_buf[rr, pl.ds(pl.multiple_of(bl, 8), 16)]
    return r * D + bl + _pcount(w, m), ncbase

  e_prev0, cb0 = _e_of(m0 - 1, jnp.int32(-(1 << 20)))

  def _group(g, carry):
    e_prev, cbase = carry

    def _one(k, kc):
      e_prev, cbase, vec = kc
      m = m0 + g * 16 + k
      e, cbase = _e_of(m, cbase)
      d = (e - e_prev).astype(jnp.float32)
      vec = jnp.where(lane == k, d, vec)
      return (e, cbase, vec)

    e_prev, cbase, vec = lax.fori_loop(
        0, 16, _one, (e_prev, cbase, zeros16))
    cnt_buf[pl.ds(g * 16, 16)] = vec
    return (e_prev, cbase)

  def _pair(t, carry):
    _half(2 * t, buf_a, gsem_a)
    carry = lax.cond(t < SEGS // 16, lambda c: _group(t, c),
                     lambda c: c, carry)
    _half(2 * t + 1, buf_b, gsem_b)
    return carry
  lax.fori_loop(0, (NBLK - 1) // 2, _pair, (e_prev0, cb0))
  _half(NBLK - 1, buf_a, gsem_a)
  pltpu.sync_copy(cnt_buf, counts_hbm.at[wid])

  plsc.subcore_barrier()
  asl = buf_a.at[pl.ds(0, CH)]
  for c in range(STRIPE // CH):
    srow = sub * STRIPE + c * CH
    pltpu.sync_copy(acc_shared.at[pl.ds(srow, CH)], asl)
    pltpu.sync_copy(asl, sums_hbm.at[pl.ds(core * M + srow, CH)])


@jax.jit
def _sc_segsum(vi, idx):
  mesh = plsc.VectorSubcoreMesh(
      core_axis_name="c", subcore_axis_name="s", num_cores=NC,
      num_subcores=NS)
  f = pl.kernel(
      _sc_body,
      out_type=(
          jax.ShapeDtypeStruct((NC * M, D), jnp.float32),
          jax.ShapeDtypeStruct((NW, SEGS), jnp.float32),
      ),
      mesh=mesh,
      scratch_types=[
          pltpu.VMEM((BLK, D), jnp.float32),
          pltpu.VMEM((BLK, D), jnp.float32),
          pltpu.VMEM((NBLK, BLK), jnp.int32),
          pltpu.VMEM((RMP,), jnp.int32),
          pltpu.VMEM((128, D), jnp.int32),
          pltpu.VMEM((SEGS,), jnp.float32),
          pltpu.VMEM_SHARED((M, D), jnp.float32),
          pltpu.SemaphoreType.DMA,
          pltpu.SemaphoreType.DMA,
          pltpu.SemaphoreType.DMA,
      ],
  )
  idx_pad = jnp.concatenate(
      [idx, jnp.full((NRP * D - N,), BIG, jnp.int32)]).reshape(NRP, D)
  rowmax = jnp.concatenate(
      [idx_pad[:, D - 1], jnp.full((RMP - NRP,), BIG, jnp.int32)])
  return f(vi, idx.reshape(NW, NBLK, BLK), idx_pad, rowmax)


def _tc_body(parts_ref, counts_ref, w1_ref, b1_ref, g1_ref, be1_ref,
             w2_ref, b2_ref, g2_ref, be2_ref, w3_ref, b3_ref, out_ref):
  seg = parts_ref[0:M] + parts_ref[M:2 * M]
  cnt = counts_ref[...]
  mean = seg / jnp.maximum(cnt, 1.0)

  h = jnp.dot(mean, w1_ref[...], preferred_element_type=jnp.float32)
  h = h + b1_ref[...]
  mu = jnp.mean(h, axis=0, keepdims=True)
  var = jnp.mean((h - mu) * (h - mu), axis=0, keepdims=True)
  h = (h - mu) / jnp.sqrt(var + 1e-5) * g1_ref[...] + be1_ref[...]
  h = jnp.maximum(h, 0.0)

  h = jnp.dot(h, w2_ref[...], preferred_element_type=jnp.float32)
  h = h + b2_ref[...]
  mu = jnp.mean(h, axis=0, keepdims=True)
  var = jnp.mean((h - mu) * (h - mu), axis=0, keepdims=True)
  h = (h - mu) / jnp.sqrt(var + 1e-5) * g2_ref[...] + be2_ref[...]
  h = jnp.maximum(h, 0.0)

  out_ref[...] = (
      jnp.dot(h, w3_ref[...], preferred_element_type=jnp.float32)
      + b3_ref[...])


@jax.jit
def _tc_mlp(parts, counts, W1, b1, g1, be1, W2, b2, g2, be2, W3, b3):
  return pl.pallas_call(
      _tc_body,
      out_shape=jax.ShapeDtypeStruct((M, 1), jnp.float32),
  )(parts, counts, W1, b1, g1, be1, W2, b2, g2, be2, W3, b3)


def kernel(vi, atom_mol_batch, W1, b1, g1, be1, W2, b2, g2, be2, W3, b3):
  idx = atom_mol_batch.astype(jnp.int32)
  parts, counts = _sc_segsum(vi, idx)
  return _tc_mlp(
      parts, counts.reshape(M, 1), W1, b1.reshape(1, D), g1.reshape(1, D),
      be1.reshape(1, D), W2, b2.reshape(1, D), g2.reshape(1, D),
      be2.reshape(1, D), W3, b3.reshape(1, 1))


# speedup vs baseline: 1.1127x; 1.0269x over previous
import jax
import jax.numpy as jnp
from jax import lax
from jax.experimental import pallas as pl
from jax.experimental.pallas import tpu as pltpu
from jax.experimental.pallas import tpu_sc as plsc

N = 320000
D = 128
M = 4096

NC = 2
NS = 16
NW = NC * NS
ROWS_PER_TILE = N // NW
BLK = 80
NBLK = ROWS_PER_TILE // BLK
STRIPE = M // NS
CH = 64
SEGS = M // NW
NR = N // D
NRP = 2560
RMP = NRP + 16
BIG = 1 << 30


def _sc_body(vi_hbm, idx_hbm, idx2d_hbm, rowmax_hbm, sums_hbm, counts_hbm,
             buf_a, buf_b, buf_c, idx_all, rowmax_buf, region_buf, cnt_buf,
             acc_shared, gsem_a, gsem_b, gsem_c, rsem_a, rsem_b, rsem_c):
  core = lax.axis_index("c")
  sub = lax.axis_index("s")
  wid = core * NS + sub
  base = wid * ROWS_PER_TILE

  zeros16 = jnp.zeros((16,), jnp.float32)

  def _zero_row(i, _):
    for j in range(D // 16):
      buf_a[i, pl.ds(j * 16, 16)] = zeros16
    return 0
  lax.fori_loop(0, BLK, _zero_row, 0)

  zsl = buf_a.at[pl.ds(0, CH)]
  for c in range(STRIPE // CH):
    pltpu.sync_copy(zsl, acc_shared.at[pl.ds(sub * STRIPE + c * CH, CH)])

  pltpu.sync_copy(idx_hbm.at[wid], idx_all)

  pltpu.async_copy(vi_hbm.at[pl.ds(base, BLK)], buf_a, gsem_a)
  pltpu.async_copy(vi_hbm.at[pl.ds(base + BLK, BLK)], buf_b, gsem_b)
  pltpu.sync_copy(rowmax_hbm, rowmax_buf)
  plsc.subcore_barrier()

  bufs = (buf_a, buf_b, buf_c)
  gsems = (gsem_a, gsem_b, gsem_c)
  rsems = (rsem_a, rsem_b, rsem_c)

  def _step(s, i):
    buf = bufs[i]
    pltpu.make_async_copy(
        vi_hbm.at[pl.ds(base + s * BLK, BLK)], buf, gsems[i]).wait()
    pltpu.async_copy(buf, acc_shared.at[idx_all.at[s]], rsems[i],
                     add=True)
    pbuf, prsem, pgsem = bufs[(i + 2) % 3], rsems[(i + 2) % 3], \
        gsems[(i + 2) % 3]

    @pl.when(s >= 1)
    def _():
      pltpu.make_async_copy(pbuf, acc_shared.at[idx_all.at[s]],
                            prsem).wait()

    @pl.when(s + 2 < NBLK)
    def _():
      pltpu.async_copy(vi_hbm.at[pl.ds(base + (s + 2) * BLK, BLK)], pbuf,
                       pgsem)

  m0 = wid * SEGS
  lane = lax.iota(jnp.int32, 16)

  def _pcount(v, m):
    t = jnp.int32(0)
    for j in range(16):
      t = t + jnp.where(v[j] <= m, 1, 0)
    return t

  def _find_row(m):
    def bs(_, lohi):
      lo, hi = lohi
      mid = (lo + hi) // 2
      v = rowmax_buf[pl.ds(pl.multiple_of(mid * 8, 8), 16)]
      go = v[0] <= m
      nlo = jnp.where(go, mid + 1, lo)
      nhi = jnp.where(go, hi, mid)
      live = lo < hi
      return (jnp.where(live, nlo, lo), jnp.where(live, nhi, hi))
    p, _ = lax.fori_loop(0, 9, bs, (jnp.int32(0), jnp.int32(NRP // 8)))
    bl = jnp.maximum(8 * (p - 1), 0)
    w = rowmax_buf[pl.ds(pl.multiple_of(bl, 8), 16)]
    return bl + _pcount(w, m)

  def _e_of(m, cbase):
    r = _find_row(m)
    need = (r < cbase) | (r >= cbase + 128)
    ncbase = jnp.where(need, jnp.minimum((r // 8) * 8, NRP - 128), cbase)

    @pl.when(need)
    def _():
      pltpu.sync_copy(
          idx2d_hbm.at[pl.ds(pl.multiple_of(ncbase, 8), 128)], region_buf)
    rr = r - ncbase

    def ibs(_, lohi):
      lo, hi = lohi
      mid = (lo + hi) // 2
      v = region_buf[rr, pl.ds(pl.multiple_of(mid * 8, 8), 16)]
      go = v[0] <= m
      nlo = jnp.where(go, mid + 1, lo)
      nhi = jnp.where(go, hi, mid)
      live = lo < hi
      return (jnp.where(live, nlo, lo), jnp.where(live, nhi, hi))
    q, _ = lax.fori_loop(0, 4, ibs, (jnp.int32(0), jnp.int32(D // 8 - 1)))
    bl = jnp.clip(8 * (q - 1), 0, D - 16)
    w = region_buf[rr, pl.ds(pl.multiple_of(bl, 8), 16)]
    return r * D + bl + _pcount(w, m), ncbase

  e_prev0, cb0 = _e_of(m0 - 1, jnp.int32(-(1 << 20)))

  def _group(g, carry):
    e_prev, cbase = carry

    def _one(k, kc):
      e_prev, cbase, vec = kc
      m = m0 + g * 16 + k
      e, cbase = _e_of(m, cbase)
      d = (e - e_prev).astype(jnp.float32)
      vec = jnp.where(lane == k, d, vec)
      return (e, cbase, vec)

    e_prev, cbase, vec = lax.fori_loop(
        0, 16, _one, (e_prev, cbase, zeros16))
    cnt_buf[pl.ds(g * 16, 16)] = vec
    return (e_prev, cbase)

  def _triple(t, carry):
    _step(3 * t, 0)
    carry = lax.cond(t < SEGS // 16, lambda c: _group(t, c),
                     lambda c: c, carry)
    _step(3 * t + 1, 1)
    _step(3 * t + 2, 2)
    return carry
  lax.fori_loop(0, NBLK // 3, _triple, (e_prev0, cb0))
  _step(NBLK - 2, 0)
  _step(NBLK - 1, 1)
  pltpu.make_async_copy(buf_b, acc_shared.at[idx_all.at[NBLK - 1]],
                        rsem_b).wait()
  pltpu.sync_copy(cnt_buf, counts_hbm.at[wid])

  plsc.subcore_barrier()
  asl = buf_a.at[pl.ds(0, CH)]
  for c in range(STRIPE // CH):
    srow = sub * STRIPE + c * CH
    pltpu.sync_copy(acc_shared.at[pl.ds(srow, CH)], asl)
    pltpu.sync_copy(asl, sums_hbm.at[pl.ds(core * M + srow, CH)])


@jax.jit
def _sc_segsum(vi, idx):
  mesh = plsc.VectorSubcoreMesh(
      core_axis_name="c", subcore_axis_name="s", num_cores=NC,
      num_subcores=NS)
  f = pl.kernel(
      _sc_body,
      out_type=(
          jax.ShapeDtypeStruct((NC * M, D), jnp.float32),
          jax.ShapeDtypeStruct((NW, SEGS), jnp.float32),
      ),
      mesh=mesh,
      scratch_types=[
          pltpu.VMEM((BLK, D), jnp.float32),
          pltpu.VMEM((BLK, D), jnp.float32),
          pltpu.VMEM((BLK, D), jnp.float32),
          pltpu.VMEM((NBLK, BLK), jnp.int32),
          pltpu.VMEM((RMP,), jnp.int32),
          pltpu.VMEM((128, D), jnp.int32),
          pltpu.VMEM((SEGS,), jnp.float32),
          pltpu.VMEM_SHARED((M, D), jnp.float32),
          pltpu.SemaphoreType.DMA,
          pltpu.SemaphoreType.DMA,
          pltpu.SemaphoreType.DMA,
          pltpu.SemaphoreType.DMA,
          pltpu.SemaphoreType.DMA,
          pltpu.SemaphoreType.DMA,
      ],
  )
  idx_pad = jnp.concatenate(
      [idx, jnp.full((NRP * D - N,), BIG, jnp.int32)]).reshape(NRP, D)
  rowmax = jnp.concatenate(
      [idx_pad[:, D - 1], jnp.full((RMP - NRP,), BIG, jnp.int32)])
  return f(vi, idx.reshape(NW, NBLK, BLK), idx_pad, rowmax)


def _tc_body(parts_ref, counts_ref, w1_ref, b1_ref, g1_ref, be1_ref,
             w2_ref, b2_ref, g2_ref, be2_ref, w3_ref, b3_ref, out_ref):
  seg = parts_ref[0:M] + parts_ref[M:2 * M]
  cnt = counts_ref[...]
  mean = seg / jnp.maximum(cnt, 1.0)

  h = jnp.dot(mean, w1_ref[...], preferred_element_type=jnp.float32)
  h = h + b1_ref[...]
  mu = jnp.mean(h, axis=0, keepdims=True)
  var = jnp.mean((h - mu) * (h - mu), axis=0, keepdims=True)
  h = (h - mu) / jnp.sqrt(var + 1e-5) * g1_ref[...] + be1_ref[...]
  h = jnp.maximum(h, 0.0)

  h = jnp.dot(h, w2_ref[...], preferred_element_type=jnp.float32)
  h = h + b2_ref[...]
  mu = jnp.mean(h, axis=0, keepdims=True)
  var = jnp.mean((h - mu) * (h - mu), axis=0, keepdims=True)
  h = (h - mu) / jnp.sqrt(var + 1e-5) * g2_ref[...] + be2_ref[...]
  h = jnp.maximum(h, 0.0)

  out_ref[...] = (
      jnp.dot(h, w3_ref[...], preferred_element_type=jnp.float32)
      + b3_ref[...])


@jax.jit
def _tc_mlp(parts, counts, W1, b1, g1, be1, W2, b2, g2, be2, W3, b3):
  return pl.pallas_call(
      _tc_body,
      out_shape=jax.ShapeDtypeStruct((M, 1), jnp.float32),
  )(parts, counts, W1, b1, g1, be1, W2, b2, g2, be2, W3, b3)


def kernel(vi, atom_mol_batch, W1, b1, g1, be1, W2, b2, g2, be2, W3, b3):
  idx = atom_mol_batch.astype(jnp.int32)
  parts, counts = _sc_segsum(vi, idx)
  return _tc_mlp(
      parts, counts.reshape(M, 1), W1, b1.reshape(1, D), g1.reshape(1, D),
      be1.reshape(1, D), W2, b2.reshape(1, D), g2.reshape(1, D),
      be2.reshape(1, D), W3, b3.reshape(1, 1))
